# baseline (speedup 1.0000x reference)
import functools

import jax
import jax.numpy as jnp
from jax import lax
from jax.experimental import pallas as pl
from jax.experimental.pallas import tpu as pltpu
from jax.experimental.pallas import tpu_sc as plsc

E = 8
D = 1024
F = 4096
T = 2048
CAP = 640
NSLOT = E * CAP
FB = 512
NC, NS = 2, 16
NW = NC * NS
ROWS_PER_TILE = NSLOT // NW
RCHUNK = 16
NBUF = 4
TOK_PER_TILE = T // NW
TCHUNK = 16

_SC_MESH = dict(core_axis_name="c", subcore_axis_name="s",
                num_cores=NC, num_subcores=NS)
_SC_PARAMS = pltpu.CompilerParams(needs_layout_passes=False)


def _router_body(h_ref, rw_ref, d1_ref, d2_ref, v1_ref, v2_ref,
                 g1_ref, g2_ref):
    x = h_ref[...]
    rw = rw_ref[...]
    logits = jnp.dot(x, rw, preferred_element_type=jnp.float32)
    m = jnp.max(logits, axis=1, keepdims=True)
    z = jnp.exp(logits - m)
    p = z / jnp.sum(z, axis=1, keepdims=True)
    lane = lax.broadcasted_iota(jnp.int32, (T, E), 1).astype(jnp.float32)
    BIG = jnp.float32(1e9)
    p1 = jnp.max(p, axis=1, keepdims=True)
    i1 = jnp.min(jnp.where(p == p1, lane, BIG), axis=1, keepdims=True)
    pm = jnp.where(lane == i1, -1.0, p)
    p2 = jnp.max(pm, axis=1, keepdims=True)
    i2 = jnp.min(jnp.where(pm == p2, lane, BIG), axis=1, keepdims=True)
    s = p1 + p2
    oh1 = lane == i1
    oh2 = lane == i2
    cnt = oh1.astype(jnp.float32) + oh2.astype(jnp.float32)
    r = lax.broadcasted_iota(jnp.int32, (128, 128), 0)
    c = lax.broadcasted_iota(jnp.int32, (128, 128), 1)
    ltri = (r > c).astype(jnp.float32)
    tot = jnp.zeros((1, E), jnp.float32)
    blocks = []
    for b in range(T // 128):
        cb = cnt[b * 128:(b + 1) * 128, :]
        blocks.append(jnp.dot(ltri, cb, preferred_element_type=jnp.float32)
                      + tot)
        tot = tot + jnp.sum(cb, axis=0, keepdims=True)
    excl = jnp.concatenate(blocks, axis=0)
    pos1 = jnp.sum(jnp.where(oh1, excl, 0.0), axis=1, keepdims=True)
    pos2 = jnp.sum(jnp.where(oh2, excl, 0.0), axis=1, keepdims=True)
    eidx = lax.broadcasted_iota(jnp.int32, (1, E), 1).astype(jnp.float32)
    free_e = jnp.min(jnp.where(tot < CAP, eidx, BIG), axis=1, keepdims=True)
    free_slot = free_e * CAP + (CAP - 1)
    val1 = pos1 < CAP
    val2 = pos2 < CAP
    d1_ref[...] = jnp.where(val1, i1 * CAP + pos1, free_slot).astype(jnp.int32)
    d2_ref[...] = jnp.where(val2, i2 * CAP + pos2, free_slot).astype(jnp.int32)
    v1_ref[...] = val1.astype(jnp.int32)
    v2_ref[...] = val2.astype(jnp.int32)
    g1_ref[...] = p1 / s
    g2_ref[...] = p2 / s


def _router(h, rw):
    col_i = jax.ShapeDtypeStruct((T, 1), jnp.int32)
    col_f = jax.ShapeDtypeStruct((T, 1), jnp.float32)
    outs = pl.pallas_call(
        _router_body,
        out_shape=(col_i, col_i, col_i, col_i, col_f, col_f),
    )(h, rw)
    return tuple(o.reshape(T) for o in outs)


def _dispatch_body(d1_hbm, d2_hbm, v1_hbm, v2_hbm, g1_hbm, g2_hbm, x_hbm,
                   xe_hbm, gs_hbm,
                   d1v, d2v, v1v, v2v, g1v, g2v, siv, gsv, shs,
                   idxs, bufs, gsem, wsem):
    cid = lax.axis_index("c")
    sid = lax.axis_index("s")

    @pl.when(sid == 0)
    def _():
        pltpu.sync_copy(d1_hbm, d1v)
        pltpu.sync_copy(d2_hbm, d2v)
        pltpu.sync_copy(v1_hbm, v1v)
        pltpu.sync_copy(v2_hbm, v2v)
        pltpu.sync_copy(g1_hbm, g1v)
        pltpu.sync_copy(g2_hbm, g2v)

        def zero(i, carry):
            sl = pl.ds(i * 16, 16)
            siv[sl] = jnp.zeros((16,), jnp.int32)
            gsv[sl] = jnp.zeros((16,), jnp.float32)
            return carry
        lax.fori_loop(0, NSLOT // 16, zero, 0)

        def scat(i, carry):
            sl = pl.ds(i * 16, 16)
            tok = lax.iota(jnp.int32, 16) + i * 16
            for dv, vv, gv in ((d1v, v1v, g1v), (d2v, v2v, g2v)):
                d = dv[sl]
                msk = vv[sl] > 0
                plsc.store_scatter(siv, [d], tok, mask=msk)
                plsc.store_scatter(gsv, [d], gv[sl], mask=msk)
            return carry
        lax.fori_loop(0, T // 16, scat, 0)

        pltpu.sync_copy(siv, shs)

        @pl.when(cid == 0)
        def _():
            pltpu.sync_copy(gsv, gs_hbm)

    plsc.subcore_barrier()

    wid = cid * NS + sid
    base = wid * ROWS_PER_TILE
    nck = ROWS_PER_TILE // RCHUNK
    for c in range(nck):
        pltpu.sync_copy(shs.at[pl.ds(base + c * RCHUNK, RCHUNK)], idxs[c])
    g = [None] * nck
    w = [None] * nck
    for c in range(NBUF):
        g[c] = pltpu.async_copy(x_hbm.at[idxs[c]], bufs[c], gsem)
    for c in range(nck):
        g[c].wait()
        w[c] = pltpu.async_copy(
            bufs[c % NBUF], xe_hbm.at[pl.ds(base + c * RCHUNK, RCHUNK)], wsem)
        if c + NBUF < nck:
            w[c].wait()
            g[c + NBUF] = pltpu.async_copy(x_hbm.at[idxs[c + NBUF]],
                                           bufs[(c + NBUF) % NBUF], gsem)
    for c in range(max(0, nck - NBUF), nck):
        w[c].wait()


def _dispatch(d1, d2, v1, v2, g1, g2, x):
    kern = pl.kernel(
        _dispatch_body,
        out_type=(jax.ShapeDtypeStruct((NSLOT, D), jnp.float32),
                  jax.ShapeDtypeStruct((NSLOT,), jnp.float32)),
        mesh=plsc.VectorSubcoreMesh(**_SC_MESH),
        compiler_params=_SC_PARAMS,
        scratch_types=[
            pltpu.VMEM((T,), jnp.int32), pltpu.VMEM((T,), jnp.int32),
            pltpu.VMEM((T,), jnp.int32), pltpu.VMEM((T,), jnp.int32),
            pltpu.VMEM((T,), jnp.float32), pltpu.VMEM((T,), jnp.float32),
            pltpu.VMEM((NSLOT,), jnp.int32), pltpu.VMEM((NSLOT,), jnp.float32),
            pltpu.VMEM_SHARED((NSLOT,), jnp.int32),
            [pltpu.VMEM((RCHUNK,), jnp.int32)
             for _ in range(ROWS_PER_TILE // RCHUNK)],
            [pltpu.VMEM((RCHUNK, D), jnp.float32) for _ in range(NBUF)],
            pltpu.SemaphoreType.DMA,
            pltpu.SemaphoreType.DMA,
        ],
    )
    return kern(d1, d2, v1, v2, g1, g2, x)


def _gemm_body(xe_ref, w1_ref, w2_ref, gs_ref, out_ref):
    f = pl.program_id(1)
    h1 = jnp.dot(xe_ref[...], w1_ref[0], preferred_element_type=jnp.float32)
    h1 = jax.nn.gelu(h1)
    y = jnp.dot(h1, w2_ref[0], preferred_element_type=jnp.float32)
    y = y * gs_ref[...]

    @pl.when(f == 0)
    def _():
        out_ref[...] = y

    @pl.when(f > 0)
    def _():
        out_ref[...] += y


def _gemm(xe, w1, w2, gs):
    gs2 = gs.reshape(NSLOT, 1)
    return pl.pallas_call(
        _gemm_body,
        grid=(E, F // FB),
        in_specs=[
            pl.BlockSpec((CAP, D), lambda e, f: (e, 0)),
            pl.BlockSpec((1, D, FB), lambda e, f: (e, 0, f)),
            pl.BlockSpec((1, FB, D), lambda e, f: (e, f, 0)),
            pl.BlockSpec((CAP, 1), lambda e, f: (e, 0)),
        ],
        out_specs=pl.BlockSpec((CAP, D), lambda e, f: (e, 0)),
        out_shape=jax.ShapeDtypeStruct((NSLOT, D), jnp.float32),
        compiler_params=pltpu.CompilerParams(
            dimension_semantics=("parallel", "arbitrary")),
    )(xe, w1, w2, gs2)


def _combine_body(x_hbm, yg_hbm, d1_hbm, d2_hbm, out_hbm,
                  d1a, d2a, d1b, d2b, y1a, y2a, xa, y1b, y2b, xb,
                  gsem, xsem, wsem):
    wid = lax.axis_index("c") * NS + lax.axis_index("s")
    tbase = wid * TOK_PER_TILE
    nck = TOK_PER_TILE // TCHUNK
    d1s = (d1a, d1b)
    d2s = (d2a, d2b)
    y1s = (y1a, y1b)
    y2s = (y2a, y2b)
    xs = (xa, xb)

    def issue(c, s):
        sl = pl.ds(tbase + c * TCHUNK, TCHUNK)
        pltpu.sync_copy(d1_hbm.at[sl], d1s[s])
        pltpu.sync_copy(d2_hbm.at[sl], d2s[s])
        return (pltpu.async_copy(yg_hbm.at[d1s[s]], y1s[s], gsem),
                pltpu.async_copy(yg_hbm.at[d2s[s]], y2s[s], gsem),
                pltpu.async_copy(x_hbm.at[sl], xs[s], xsem))

    dma = [None] * nck
    w = [None] * nck
    dma[0] = issue(0, 0)
    for c in range(nck):
        for t in dma[c]:
            t.wait()
        if c + 1 < nck:
            if c - 1 >= 0:
                w[c - 1].wait()
            dma[c + 1] = issue(c + 1, (c + 1) % 2)
        s = c % 2
        xv, y1, y2 = xs[s], y1s[s], y2s[s]
        for t in range(TCHUNK):
            def add(j, carry, t=t, xv=xv, y1=y1, y2=y2):
                for u in range(4):
                    sl = pl.ds(j * 64 + u * 16, 16)
                    xv[t, sl] = xv[t, sl] + y1[t, sl] + y2[t, sl]
                return carry
            lax.fori_loop(0, D // 64, add, 0)
        w[c] = pltpu.async_copy(xv, out_hbm.at[pl.ds(tbase + c * TCHUNK,
                                                     TCHUNK)], wsem)
    w[nck - 2].wait()
    w[nck - 1].wait()


def _combine(x, yg, d1, d2):
    kern = pl.kernel(
        _combine_body,
        out_type=jax.ShapeDtypeStruct((T, D), jnp.float32),
        mesh=plsc.VectorSubcoreMesh(**_SC_MESH),
        compiler_params=_SC_PARAMS,
        scratch_types=[
            pltpu.VMEM((TCHUNK,), jnp.int32),
            pltpu.VMEM((TCHUNK,), jnp.int32),
            pltpu.VMEM((TCHUNK,), jnp.int32),
            pltpu.VMEM((TCHUNK,), jnp.int32),
            pltpu.VMEM((TCHUNK, D), jnp.float32),
            pltpu.VMEM((TCHUNK, D), jnp.float32),
            pltpu.VMEM((TCHUNK, D), jnp.float32),
            pltpu.VMEM((TCHUNK, D), jnp.float32),
            pltpu.VMEM((TCHUNK, D), jnp.float32),
            pltpu.VMEM((TCHUNK, D), jnp.float32),
            pltpu.SemaphoreType.DMA,
            pltpu.SemaphoreType.DMA,
            pltpu.SemaphoreType.DMA,
        ],
    )
    return kern(x, yg, d1, d2)


def _moe_layer(h, rw, w1l, w2l):
    d1, d2, v1, v2, g1, g2 = _router(h, rw)
    xe, gs = _dispatch(d1, d2, v1, v2, g1, g2, h)
    yg = _gemm(xe, w1l, w2l, gs)
    return _combine(h, yg, d1, d2)


def kernel(x, router_w, w1, w2):
    h = x
    for l in range(2):
        h = _moe_layer(h, router_w[l], w1[l], w2[l])
    return h


# speedup vs baseline: 1.1248x; 1.1248x over previous
import functools

import jax
import jax.numpy as jnp
from jax import lax
from jax.experimental import pallas as pl
from jax.experimental.pallas import tpu as pltpu
from jax.experimental.pallas import tpu_sc as plsc

E = 8
D = 1024
F = 4096
T = 2048
CAP = 640
NSLOT = E * CAP
FB = 512
NC, NS = 2, 16
NW = NC * NS
ROWS_PER_TILE = NSLOT // NW
RCHUNK = 16
NBUF = 4
TOK_PER_TILE = T // NW
TCHUNK = 16

_SC_MESH = dict(core_axis_name="c", subcore_axis_name="s",
                num_cores=NC, num_subcores=NS)
_SC_PARAMS = pltpu.CompilerParams(needs_layout_passes=False)


def _router_body(h_ref, rw_ref, d1_ref, d2_ref, v1_ref, v2_ref,
                 g1_ref, g2_ref):
    x = h_ref[...]
    rw = rw_ref[...]
    logits = jnp.dot(x, rw, preferred_element_type=jnp.float32)
    m = jnp.max(logits, axis=1, keepdims=True)
    z = jnp.exp(logits - m)
    p = z / jnp.sum(z, axis=1, keepdims=True)
    lane = lax.broadcasted_iota(jnp.int32, (T, E), 1).astype(jnp.float32)
    BIG = jnp.float32(1e9)
    p1 = jnp.max(p, axis=1, keepdims=True)
    i1 = jnp.min(jnp.where(p == p1, lane, BIG), axis=1, keepdims=True)
    pm = jnp.where(lane == i1, -1.0, p)
    p2 = jnp.max(pm, axis=1, keepdims=True)
    i2 = jnp.min(jnp.where(pm == p2, lane, BIG), axis=1, keepdims=True)
    s = p1 + p2
    oh1 = lane == i1
    oh2 = lane == i2
    cnt = oh1.astype(jnp.float32) + oh2.astype(jnp.float32)
    r = lax.broadcasted_iota(jnp.int32, (128, 128), 0)
    c = lax.broadcasted_iota(jnp.int32, (128, 128), 1)
    ltri = (r > c).astype(jnp.float32)
    tot = jnp.zeros((1, E), jnp.float32)
    blocks = []
    for b in range(T // 128):
        cb = cnt[b * 128:(b + 1) * 128, :]
        blocks.append(jnp.dot(ltri, cb, preferred_element_type=jnp.float32)
                      + tot)
        tot = tot + jnp.sum(cb, axis=0, keepdims=True)
    excl = jnp.concatenate(blocks, axis=0)
    pos1 = jnp.sum(jnp.where(oh1, excl, 0.0), axis=1, keepdims=True)
    pos2 = jnp.sum(jnp.where(oh2, excl, 0.0), axis=1, keepdims=True)
    eidx = lax.broadcasted_iota(jnp.int32, (1, E), 1).astype(jnp.float32)
    free_e = jnp.min(jnp.where(tot < CAP, eidx, BIG), axis=1, keepdims=True)
    free_slot = free_e * CAP + (CAP - 1)
    val1 = pos1 < CAP
    val2 = pos2 < CAP
    d1_ref[...] = jnp.where(val1, i1 * CAP + pos1, free_slot).astype(jnp.int32)
    d2_ref[...] = jnp.where(val2, i2 * CAP + pos2, free_slot).astype(jnp.int32)
    v1_ref[...] = val1.astype(jnp.int32)
    v2_ref[...] = val2.astype(jnp.int32)
    g1_ref[...] = p1 / s
    g2_ref[...] = p2 / s


def _router(h, rw):
    col_i = jax.ShapeDtypeStruct((T, 1), jnp.int32)
    col_f = jax.ShapeDtypeStruct((T, 1), jnp.float32)
    outs = pl.pallas_call(
        _router_body,
        out_shape=(col_i, col_i, col_i, col_i, col_f, col_f),
    )(h, rw)
    return tuple(o.reshape(T) for o in outs)


def _build_body(d1_hbm, d2_hbm, v1_hbm, v2_hbm, g1_hbm, g2_hbm,
                si_hbm, gs_hbm,
                d1v, d2v, v1v, v2v, g1v, g2v, siv, gsv):
    cid = lax.axis_index("c")
    sid = lax.axis_index("s")

    @pl.when((sid == 0) & (cid == 0))
    def _():
        pltpu.sync_copy(d1_hbm, d1v)
        pltpu.sync_copy(d2_hbm, d2v)
        pltpu.sync_copy(v1_hbm, v1v)
        pltpu.sync_copy(v2_hbm, v2v)
        pltpu.sync_copy(g1_hbm, g1v)
        pltpu.sync_copy(g2_hbm, g2v)

        def zero(i, carry):
            sl = pl.ds(i * 16, 16)
            siv[sl] = jnp.zeros((16,), jnp.int32)
            gsv[sl] = jnp.zeros((16,), jnp.float32)
            return carry
        lax.fori_loop(0, NSLOT // 16, zero, 0)

        def scat(i, carry):
            sl = pl.ds(i * 16, 16)
            tok = lax.iota(jnp.int32, 16) + i * 16
            for dv, vv, gv in ((d1v, v1v, g1v), (d2v, v2v, g2v)):
                d = dv[sl]
                msk = vv[sl] > 0
                plsc.store_scatter(siv, [d], tok, mask=msk)
                plsc.store_scatter(gsv, [d], gv[sl], mask=msk)
            return carry
        lax.fori_loop(0, T // 16, scat, 0)

        pltpu.sync_copy(siv, si_hbm)
        pltpu.sync_copy(gsv, gs_hbm)


def _build_idx(d1, d2, v1, v2, g1, g2):
    kern = pl.kernel(
        _build_body,
        out_type=(jax.ShapeDtypeStruct((NSLOT,), jnp.int32),
                  jax.ShapeDtypeStruct((NSLOT,), jnp.float32)),
        mesh=plsc.VectorSubcoreMesh(**_SC_MESH),
        compiler_params=_SC_PARAMS,
        scratch_types=[
            pltpu.VMEM((T,), jnp.int32), pltpu.VMEM((T,), jnp.int32),
            pltpu.VMEM((T,), jnp.int32), pltpu.VMEM((T,), jnp.int32),
            pltpu.VMEM((T,), jnp.float32), pltpu.VMEM((T,), jnp.float32),
            pltpu.VMEM((NSLOT,), jnp.int32), pltpu.VMEM((NSLOT,), jnp.float32),
        ],
    )
    return kern(d1, d2, v1, v2, g1, g2)


KB = 256


def _gemm_body(si_ref, x_ref, w1_ref, w2_ref, gs_ref, out_ref, xe_s):
    f = pl.program_id(1)

    @pl.when(f == 0)
    def _():
        si_col = si_ref[...]
        acc = jnp.zeros((CAP, D), jnp.float32)
        for kb in range(T // KB):
            tok = lax.broadcasted_iota(jnp.int32, (CAP, KB), 1) + kb * KB
            ce = (si_col == tok).astype(jnp.float32)
            acc = acc + jnp.dot(ce, x_ref[kb * KB:(kb + 1) * KB, :],
                                preferred_element_type=jnp.float32)
        xe_s[...] = acc

    h1 = jnp.dot(xe_s[...], w1_ref[0], preferred_element_type=jnp.float32)
    h1 = jax.nn.gelu(h1)
    y = jnp.dot(h1, w2_ref[0], preferred_element_type=jnp.float32)
    y = y * gs_ref[...]

    @pl.when(f == 0)
    def _():
        out_ref[...] = y

    @pl.when(f > 0)
    def _():
        out_ref[...] += y


def _gemm(si, x, w1, w2, gs):
    si2 = si.reshape(NSLOT, 1)
    gs2 = gs.reshape(NSLOT, 1)
    return pl.pallas_call(
        _gemm_body,
        grid=(E, F // FB),
        in_specs=[
            pl.BlockSpec((CAP, 1), lambda e, f: (e, 0)),
            pl.BlockSpec((T, D), lambda e, f: (0, 0)),
            pl.BlockSpec((1, D, FB), lambda e, f: (e, 0, f)),
            pl.BlockSpec((1, FB, D), lambda e, f: (e, f, 0)),
            pl.BlockSpec((CAP, 1), lambda e, f: (e, 0)),
        ],
        out_specs=pl.BlockSpec((CAP, D), lambda e, f: (e, 0)),
        out_shape=jax.ShapeDtypeStruct((NSLOT, D), jnp.float32),
        scratch_shapes=[pltpu.VMEM((CAP, D), jnp.float32)],
        compiler_params=pltpu.CompilerParams(
            dimension_semantics=("parallel", "arbitrary")),
    )(si2, x, w1, w2, gs2)


def _combine_body(x_hbm, yg_hbm, d1_hbm, d2_hbm, out_hbm,
                  d1a, d2a, d1b, d2b, y1a, y2a, xa, y1b, y2b, xb,
                  gsem, xsem, wsem):
    wid = lax.axis_index("c") * NS + lax.axis_index("s")
    tbase = wid * TOK_PER_TILE
    nck = TOK_PER_TILE // TCHUNK
    d1s = (d1a, d1b)
    d2s = (d2a, d2b)
    y1s = (y1a, y1b)
    y2s = (y2a, y2b)
    xs = (xa, xb)

    def issue(c, s):
        sl = pl.ds(tbase + c * TCHUNK, TCHUNK)
        pltpu.sync_copy(d1_hbm.at[sl], d1s[s])
        pltpu.sync_copy(d2_hbm.at[sl], d2s[s])
        return (pltpu.async_copy(yg_hbm.at[d1s[s]], y1s[s], gsem),
                pltpu.async_copy(yg_hbm.at[d2s[s]], y2s[s], gsem),
                pltpu.async_copy(x_hbm.at[sl], xs[s], xsem))

    dma = [None] * nck
    w = [None] * nck
    dma[0] = issue(0, 0)
    for c in range(nck):
        for t in dma[c]:
            t.wait()
        if c + 1 < nck:
            if c - 1 >= 0:
                w[c - 1].wait()
            dma[c + 1] = issue(c + 1, (c + 1) % 2)
        s = c % 2
        xv, y1, y2 = xs[s], y1s[s], y2s[s]
        for t in range(TCHUNK):
            def add(j, carry, t=t, xv=xv, y1=y1, y2=y2):
                for u in range(4):
                    sl = pl.ds(j * 64 + u * 16, 16)
                    xv[t, sl] = xv[t, sl] + y1[t, sl] + y2[t, sl]
                return carry
            lax.fori_loop(0, D // 64, add, 0)
        w[c] = pltpu.async_copy(xv, out_hbm.at[pl.ds(tbase + c * TCHUNK,
                                                     TCHUNK)], wsem)
    w[nck - 2].wait()
    w[nck - 1].wait()


def _combine(x, yg, d1, d2):
    kern = pl.kernel(
        _combine_body,
        out_type=jax.ShapeDtypeStruct((T, D), jnp.float32),
        mesh=plsc.VectorSubcoreMesh(**_SC_MESH),
        compiler_params=_SC_PARAMS,
        scratch_types=[
            pltpu.VMEM((TCHUNK,), jnp.int32),
            pltpu.VMEM((TCHUNK,), jnp.int32),
            pltpu.VMEM((TCHUNK,), jnp.int32),
            pltpu.VMEM((TCHUNK,), jnp.int32),
            pltpu.VMEM((TCHUNK, D), jnp.float32),
            pltpu.VMEM((TCHUNK, D), jnp.float32),
            pltpu.VMEM((TCHUNK, D), jnp.float32),
            pltpu.VMEM((TCHUNK, D), jnp.float32),
            pltpu.VMEM((TCHUNK, D), jnp.float32),
            pltpu.VMEM((TCHUNK, D), jnp.float32),
            pltpu.SemaphoreType.DMA,
            pltpu.SemaphoreType.DMA,
            pltpu.SemaphoreType.DMA,
        ],
    )
    return kern(x, yg, d1, d2)


def _moe_layer(h, rw, w1l, w2l):
    d1, d2, v1, v2, g1, g2 = _router(h, rw)
    si, gs = _build_idx(d1, d2, v1, v2, g1, g2)
    yg = _gemm(si, h, w1l, w2l, gs)
    return _combine(h, yg, d1, d2)


def kernel(x, router_w, w1, w2):
    h = x
    for l in range(2):
        h = _moe_layer(h, router_w[l], w1[l], w2[l])
    return h


# speedup vs baseline: 1.8677x; 1.6605x over previous
import functools

import jax
import jax.numpy as jnp
from jax import lax
from jax.experimental import pallas as pl
from jax.experimental.pallas import tpu as pltpu
from jax.experimental.pallas import tpu_sc as plsc

E = 8
D = 1024
F = 4096
T = 2048
CAP = 640
NSLOT = E * CAP
FB = 512
NC, NS = 2, 16
NW = NC * NS
ROWS_PER_TILE = NSLOT // NW
RCHUNK = 16
NBUF = 4
TOK_PER_TILE = T // NW
TCHUNK = 16

_SC_MESH = dict(core_axis_name="c", subcore_axis_name="s",
                num_cores=NC, num_subcores=NS)
_SC_PARAMS = pltpu.CompilerParams(needs_layout_passes=False)


def _router_body(h_ref, rw_ref, d1_ref, d2_ref, v1_ref, v2_ref,
                 g1_ref, g2_ref):
    x = h_ref[...]
    rw = rw_ref[0]
    logits = jnp.dot(x, rw, preferred_element_type=jnp.float32)
    m = jnp.max(logits, axis=1, keepdims=True)
    z = jnp.exp(logits - m)
    p = z / jnp.sum(z, axis=1, keepdims=True)
    lane = lax.broadcasted_iota(jnp.int32, (T, E), 1).astype(jnp.float32)
    BIG = jnp.float32(1e9)
    p1 = jnp.max(p, axis=1, keepdims=True)
    i1 = jnp.min(jnp.where(p == p1, lane, BIG), axis=1, keepdims=True)
    pm = jnp.where(lane == i1, -1.0, p)
    p2 = jnp.max(pm, axis=1, keepdims=True)
    i2 = jnp.min(jnp.where(pm == p2, lane, BIG), axis=1, keepdims=True)
    s = p1 + p2
    oh1 = lane == i1
    oh2 = lane == i2
    cnt = oh1.astype(jnp.float32) + oh2.astype(jnp.float32)
    r = lax.broadcasted_iota(jnp.int32, (128, 128), 0)
    c = lax.broadcasted_iota(jnp.int32, (128, 128), 1)
    ltri = (r > c).astype(jnp.float32)
    tot = jnp.zeros((1, E), jnp.float32)
    blocks = []
    for b in range(T // 128):
        cb = cnt[b * 128:(b + 1) * 128, :]
        blocks.append(jnp.dot(ltri, cb, preferred_element_type=jnp.float32)
                      + tot)
        tot = tot + jnp.sum(cb, axis=0, keepdims=True)
    excl = jnp.concatenate(blocks, axis=0)
    pos1 = jnp.sum(jnp.where(oh1, excl, 0.0), axis=1, keepdims=True)
    pos2 = jnp.sum(jnp.where(oh2, excl, 0.0), axis=1, keepdims=True)
    eidx = lax.broadcasted_iota(jnp.int32, (1, E), 1).astype(jnp.float32)
    free_e = jnp.min(jnp.where(tot < CAP, eidx, BIG), axis=1, keepdims=True)
    free_slot = free_e * CAP + (CAP - 1)
    val1 = pos1 < CAP
    val2 = pos2 < CAP
    d1_ref[...] = jnp.where(val1, i1 * CAP + pos1, free_slot).astype(jnp.int32)
    d2_ref[...] = jnp.where(val2, i2 * CAP + pos2, free_slot).astype(jnp.int32)
    v1_ref[...] = val1.astype(jnp.int32)
    v2_ref[...] = val2.astype(jnp.int32)
    g1_ref[...] = p1 / s
    g2_ref[...] = p2 / s


def _router(h, rw_full, l):
    col_i = jax.ShapeDtypeStruct((T, 1), jnp.int32)
    col_f = jax.ShapeDtypeStruct((T, 1), jnp.float32)
    outs = pl.pallas_call(
        _router_body,
        grid=(1,),
        in_specs=[
            pl.BlockSpec((T, D), lambda i: (0, 0)),
            pl.BlockSpec((1, D, E), lambda i: (l, 0, 0)),
        ],
        out_specs=tuple(pl.BlockSpec((T, 1), lambda i: (0, 0))
                        for _ in range(6)),
        out_shape=(col_i, col_i, col_i, col_i, col_f, col_f),
    )(h, rw_full)
    return tuple(o.reshape(T) for o in outs)


def _build_body(d1_hbm, d2_hbm, v1_hbm, v2_hbm, g1_hbm, g2_hbm,
                si_hbm, gs_hbm,
                d1v, d2v, v1v, v2v, g1v, g2v, siv, gsv):
    cid = lax.axis_index("c")
    sid = lax.axis_index("s")

    @pl.when((sid == 0) & (cid == 0))
    def _():
        pltpu.sync_copy(d1_hbm, d1v)
        pltpu.sync_copy(d2_hbm, d2v)
        pltpu.sync_copy(v1_hbm, v1v)
        pltpu.sync_copy(v2_hbm, v2v)
        pltpu.sync_copy(g1_hbm, g1v)
        pltpu.sync_copy(g2_hbm, g2v)

        def zero(i, carry):
            sl = pl.ds(i * 16, 16)
            siv[sl] = jnp.zeros((16,), jnp.int32)
            gsv[sl] = jnp.zeros((16,), jnp.float32)
            return carry
        lax.fori_loop(0, NSLOT // 16, zero, 0)

        def scat(i, carry):
            sl = pl.ds(i * 16, 16)
            tok = lax.iota(jnp.int32, 16) + i * 16
            for dv, vv, gv in ((d1v, v1v, g1v), (d2v, v2v, g2v)):
                d = dv[sl]
                msk = vv[sl] > 0
                plsc.store_scatter(siv, [d], tok, mask=msk)
                plsc.store_scatter(gsv, [d], gv[sl], mask=msk)
            return carry
        lax.fori_loop(0, T // 16, scat, 0)

        pltpu.sync_copy(siv, si_hbm)
        pltpu.sync_copy(gsv, gs_hbm)


def _build_idx(d1, d2, v1, v2, g1, g2):
    kern = pl.kernel(
        _build_body,
        out_type=(jax.ShapeDtypeStruct((NSLOT,), jnp.int32),
                  jax.ShapeDtypeStruct((NSLOT,), jnp.float32)),
        mesh=plsc.VectorSubcoreMesh(**_SC_MESH),
        compiler_params=_SC_PARAMS,
        scratch_types=[
            pltpu.VMEM((T,), jnp.int32), pltpu.VMEM((T,), jnp.int32),
            pltpu.VMEM((T,), jnp.int32), pltpu.VMEM((T,), jnp.int32),
            pltpu.VMEM((T,), jnp.float32), pltpu.VMEM((T,), jnp.float32),
            pltpu.VMEM((NSLOT,), jnp.int32), pltpu.VMEM((NSLOT,), jnp.float32),
        ],
    )
    return kern(d1, d2, v1, v2, g1, g2)


KB = 256


def _gemm_body(si_ref, x_ref, w1_ref, w2_ref, gs_ref, out_ref, xe_s):
    f = pl.program_id(1)

    @pl.when(f == 0)
    def _():
        si_col = si_ref[...]
        acc = jnp.zeros((CAP, D), jnp.float32)
        for kb in range(T // KB):
            tok = lax.broadcasted_iota(jnp.int32, (CAP, KB), 1) + kb * KB
            ce = (si_col == tok).astype(jnp.float32)
            acc = acc + jnp.dot(ce, x_ref[kb * KB:(kb + 1) * KB, :],
                                preferred_element_type=jnp.float32)
        xe_s[...] = acc

    h1 = jnp.dot(xe_s[...], w1_ref[0, 0], preferred_element_type=jnp.float32)
    h1 = jax.nn.gelu(h1)
    y = jnp.dot(h1, w2_ref[0, 0], preferred_element_type=jnp.float32)
    y = y * gs_ref[...]

    @pl.when(f == 0)
    def _():
        out_ref[...] = y

    @pl.when(f > 0)
    def _():
        out_ref[...] += y


def _gemm(si, x, w1, w2, gs, l):
    si2 = si.reshape(NSLOT, 1)
    gs2 = gs.reshape(NSLOT, 1)
    return pl.pallas_call(
        _gemm_body,
        grid=(E, F // FB),
        in_specs=[
            pl.BlockSpec((CAP, 1), lambda e, f: (e, 0)),
            pl.BlockSpec((T, D), lambda e, f: (0, 0)),
            pl.BlockSpec((1, 1, D, FB), lambda e, f: (l, e, 0, f)),
            pl.BlockSpec((1, 1, FB, D), lambda e, f: (l, e, f, 0)),
            pl.BlockSpec((CAP, 1), lambda e, f: (e, 0)),
        ],
        out_specs=pl.BlockSpec((CAP, D), lambda e, f: (e, 0)),
        out_shape=jax.ShapeDtypeStruct((NSLOT, D), jnp.float32),
        scratch_shapes=[pltpu.VMEM((CAP, D), jnp.float32)],
        compiler_params=pltpu.CompilerParams(
            dimension_semantics=("parallel", "arbitrary")),
    )(si2, x, w1, w2, gs2)


def _combine_body(x_hbm, yg_hbm, d1_hbm, d2_hbm, out_hbm,
                  d1a, d2a, d1b, d2b, y1a, y2a, xa, y1b, y2b, xb,
                  gsem, xsem, wsem):
    wid = lax.axis_index("c") * NS + lax.axis_index("s")
    tbase = wid * TOK_PER_TILE
    nck = TOK_PER_TILE // TCHUNK
    d1s = (d1a, d1b)
    d2s = (d2a, d2b)
    y1s = (y1a, y1b)
    y2s = (y2a, y2b)
    xs = (xa, xb)

    def issue(c, s):
        sl = pl.ds(tbase + c * TCHUNK, TCHUNK)
        pltpu.sync_copy(d1_hbm.at[sl], d1s[s])
        pltpu.sync_copy(d2_hbm.at[sl], d2s[s])
        return (pltpu.async_copy(yg_hbm.at[d1s[s]], y1s[s], gsem),
                pltpu.async_copy(yg_hbm.at[d2s[s]], y2s[s], gsem),
                pltpu.async_copy(x_hbm.at[sl], xs[s], xsem))

    dma = [None] * nck
    w = [None] * nck
    dma[0] = issue(0, 0)
    for c in range(nck):
        for t in dma[c]:
            t.wait()
        if c + 1 < nck:
            if c - 1 >= 0:
                w[c - 1].wait()
            dma[c + 1] = issue(c + 1, (c + 1) % 2)
        s = c % 2
        xv, y1, y2 = xs[s], y1s[s], y2s[s]
        for t in range(TCHUNK):
            def add(j, carry, t=t, xv=xv, y1=y1, y2=y2):
                for u in range(4):
                    sl = pl.ds(j * 64 + u * 16, 16)
                    xv[t, sl] = xv[t, sl] + y1[t, sl] + y2[t, sl]
                return carry
            lax.fori_loop(0, D // 64, add, 0)
        w[c] = pltpu.async_copy(xv, out_hbm.at[pl.ds(tbase + c * TCHUNK,
                                                     TCHUNK)], wsem)
    w[nck - 2].wait()
    w[nck - 1].wait()


def _combine(x, yg, d1, d2):
    kern = pl.kernel(
        _combine_body,
        out_type=jax.ShapeDtypeStruct((T, D), jnp.float32),
        mesh=plsc.VectorSubcoreMesh(**_SC_MESH),
        compiler_params=_SC_PARAMS,
        scratch_types=[
            pltpu.VMEM((TCHUNK,), jnp.int32),
            pltpu.VMEM((TCHUNK,), jnp.int32),
            pltpu.VMEM((TCHUNK,), jnp.int32),
            pltpu.VMEM((TCHUNK,), jnp.int32),
            pltpu.VMEM((TCHUNK, D), jnp.float32),
            pltpu.VMEM((TCHUNK, D), jnp.float32),
            pltpu.VMEM((TCHUNK, D), jnp.float32),
            pltpu.VMEM((TCHUNK, D), jnp.float32),
            pltpu.VMEM((TCHUNK, D), jnp.float32),
            pltpu.VMEM((TCHUNK, D), jnp.float32),
            pltpu.SemaphoreType.DMA,
            pltpu.SemaphoreType.DMA,
            pltpu.SemaphoreType.DMA,
        ],
    )
    return kern(x, yg, d1, d2)


def _moe_layer(h, router_w, w1, w2, l):
    d1, d2, v1, v2, g1, g2 = _router(h, router_w, l)
    si, gs = _build_idx(d1, d2, v1, v2, g1, g2)
    yg = _gemm(si, h, w1, w2, gs, l)
    return _combine(h, yg, d1, d2)


def kernel(x, router_w, w1, w2):
    h = x
    for l in range(2):
        h = _moe_layer(h, router_w, w1, w2, l)
    return h


# speedup vs baseline: 2.0643x; 1.1052x over previous
import functools

import jax
import jax.numpy as jnp
from jax import lax
from jax.experimental import pallas as pl
from jax.experimental.pallas import tpu as pltpu
from jax.experimental.pallas import tpu_sc as plsc

E = 8
D = 1024
F = 4096
T = 2048
CAP = 640
NSLOT = E * CAP
FB = 1024
NC, NS = 2, 16
NW = NC * NS
ROWS_PER_TILE = NSLOT // NW
RCHUNK = 16
NBUF = 4
TOK_PER_TILE = T // NW
TCHUNK = 16

_SC_MESH = dict(core_axis_name="c", subcore_axis_name="s",
                num_cores=NC, num_subcores=NS)
_SC_PARAMS = pltpu.CompilerParams(needs_layout_passes=False)


def _router_body(h_ref, rw_ref, d1_ref, d2_ref, v1_ref, v2_ref,
                 g1_ref, g2_ref):
    x = h_ref[...]
    rw = rw_ref[0]
    logits = jnp.dot(x, rw, preferred_element_type=jnp.float32)
    m = jnp.max(logits, axis=1, keepdims=True)
    z = jnp.exp(logits - m)
    p = z / jnp.sum(z, axis=1, keepdims=True)
    lane = lax.broadcasted_iota(jnp.int32, (T, E), 1).astype(jnp.float32)
    BIG = jnp.float32(1e9)
    p1 = jnp.max(p, axis=1, keepdims=True)
    i1 = jnp.min(jnp.where(p == p1, lane, BIG), axis=1, keepdims=True)
    pm = jnp.where(lane == i1, -1.0, p)
    p2 = jnp.max(pm, axis=1, keepdims=True)
    i2 = jnp.min(jnp.where(pm == p2, lane, BIG), axis=1, keepdims=True)
    s = p1 + p2
    oh1 = lane == i1
    oh2 = lane == i2
    cnt = oh1.astype(jnp.float32) + oh2.astype(jnp.float32)
    r = lax.broadcasted_iota(jnp.int32, (128, 128), 0)
    c = lax.broadcasted_iota(jnp.int32, (128, 128), 1)
    ltri = (r > c).astype(jnp.float32)
    tot = jnp.zeros((1, E), jnp.float32)
    blocks = []
    for b in range(T // 128):
        cb = cnt[b * 128:(b + 1) * 128, :]
        blocks.append(jnp.dot(ltri, cb, preferred_element_type=jnp.float32)
                      + tot)
        tot = tot + jnp.sum(cb, axis=0, keepdims=True)
    excl = jnp.concatenate(blocks, axis=0)
    pos1 = jnp.sum(jnp.where(oh1, excl, 0.0), axis=1, keepdims=True)
    pos2 = jnp.sum(jnp.where(oh2, excl, 0.0), axis=1, keepdims=True)
    eidx = lax.broadcasted_iota(jnp.int32, (1, E), 1).astype(jnp.float32)
    free_e = jnp.min(jnp.where(tot < CAP, eidx, BIG), axis=1, keepdims=True)
    free_slot = free_e * CAP + (CAP - 1)
    val1 = pos1 < CAP
    val2 = pos2 < CAP
    d1_ref[...] = jnp.where(val1, i1 * CAP + pos1, free_slot).astype(jnp.int32)
    d2_ref[...] = jnp.where(val2, i2 * CAP + pos2, free_slot).astype(jnp.int32)
    v1_ref[...] = val1.astype(jnp.int32)
    v2_ref[...] = val2.astype(jnp.int32)
    g1_ref[...] = p1 / s
    g2_ref[...] = p2 / s


def _router(h, rw_full, l):
    col_i = jax.ShapeDtypeStruct((T, 1), jnp.int32)
    col_f = jax.ShapeDtypeStruct((T, 1), jnp.float32)
    outs = pl.pallas_call(
        _router_body,
        grid=(1,),
        in_specs=[
            pl.BlockSpec((T, D), lambda i: (0, 0)),
            pl.BlockSpec((1, D, E), lambda i: (l, 0, 0)),
        ],
        out_specs=tuple(pl.BlockSpec((T, 1), lambda i: (0, 0))
                        for _ in range(6)),
        out_shape=(col_i, col_i, col_i, col_i, col_f, col_f),
    )(h, rw_full)
    return tuple(o.reshape(T) for o in outs)


def _build_body(d1_hbm, d2_hbm, v1_hbm, v2_hbm, g1_hbm, g2_hbm,
                si_hbm, gs_hbm,
                d1v, d2v, v1v, v2v, g1v, g2v, siv, gsv):
    cid = lax.axis_index("c")
    sid = lax.axis_index("s")

    @pl.when((sid == 0) & (cid == 0))
    def _():
        pltpu.sync_copy(d1_hbm, d1v)
        pltpu.sync_copy(d2_hbm, d2v)
        pltpu.sync_copy(v1_hbm, v1v)
        pltpu.sync_copy(v2_hbm, v2v)
        pltpu.sync_copy(g1_hbm, g1v)
        pltpu.sync_copy(g2_hbm, g2v)

        def zero(i, carry):
            sl = pl.ds(i * 16, 16)
            siv[sl] = jnp.zeros((16,), jnp.int32)
            gsv[sl] = jnp.zeros((16,), jnp.float32)
            return carry
        lax.fori_loop(0, NSLOT // 16, zero, 0)

        def scat(i, carry):
            sl = pl.ds(i * 16, 16)
            tok = lax.iota(jnp.int32, 16) + i * 16
            for dv, vv, gv in ((d1v, v1v, g1v), (d2v, v2v, g2v)):
                d = dv[sl]
                msk = vv[sl] > 0
                plsc.store_scatter(siv, [d], tok, mask=msk)
                plsc.store_scatter(gsv, [d], gv[sl], mask=msk)
            return carry
        lax.fori_loop(0, T // 16, scat, 0)

        pltpu.sync_copy(siv, si_hbm)
        pltpu.sync_copy(gsv, gs_hbm)


def _build_idx(d1, d2, v1, v2, g1, g2):
    kern = pl.kernel(
        _build_body,
        out_type=(jax.ShapeDtypeStruct((NSLOT,), jnp.int32),
                  jax.ShapeDtypeStruct((NSLOT,), jnp.float32)),
        mesh=plsc.VectorSubcoreMesh(**_SC_MESH),
        compiler_params=_SC_PARAMS,
        scratch_types=[
            pltpu.VMEM((T,), jnp.int32), pltpu.VMEM((T,), jnp.int32),
            pltpu.VMEM((T,), jnp.int32), pltpu.VMEM((T,), jnp.int32),
            pltpu.VMEM((T,), jnp.float32), pltpu.VMEM((T,), jnp.float32),
            pltpu.VMEM((NSLOT,), jnp.int32), pltpu.VMEM((NSLOT,), jnp.float32),
        ],
    )
    return kern(d1, d2, v1, v2, g1, g2)


KB = 256


def _gemm_body(si_ref, x_ref, w1_ref, w2_ref, gs_ref, out_ref, xe_s):
    f = pl.program_id(1)

    @pl.when(f == 0)
    def _():
        si_col = si_ref[...]
        acc = jnp.zeros((CAP, D), jnp.float32)
        for kb in range(T // KB):
            tok = lax.broadcasted_iota(jnp.int32, (CAP, KB), 1) + kb * KB
            ce = (si_col == tok).astype(jnp.float32)
            acc = acc + jnp.dot(ce, x_ref[kb * KB:(kb + 1) * KB, :],
                                preferred_element_type=jnp.float32)
        xe_s[...] = acc

    h1 = jnp.dot(xe_s[...], w1_ref[0, 0], preferred_element_type=jnp.float32)
    h1 = jax.nn.gelu(h1)
    y = jnp.dot(h1, w2_ref[0, 0], preferred_element_type=jnp.float32)
    y = y * gs_ref[...]

    @pl.when(f == 0)
    def _():
        out_ref[...] = y

    @pl.when(f > 0)
    def _():
        out_ref[...] += y


def _gemm(si, x, w1, w2, gs, l):
    si2 = si.reshape(NSLOT, 1)
    gs2 = gs.reshape(NSLOT, 1)
    return pl.pallas_call(
        _gemm_body,
        grid=(E, F // FB),
        in_specs=[
            pl.BlockSpec((CAP, 1), lambda e, f: (e, 0)),
            pl.BlockSpec((T, D), lambda e, f: (0, 0)),
            pl.BlockSpec((1, 1, D, FB), lambda e, f: (l, e, 0, f)),
            pl.BlockSpec((1, 1, FB, D), lambda e, f: (l, e, f, 0)),
            pl.BlockSpec((CAP, 1), lambda e, f: (e, 0)),
        ],
        out_specs=pl.BlockSpec((CAP, D), lambda e, f: (e, 0)),
        out_shape=jax.ShapeDtypeStruct((NSLOT, D), jnp.float32),
        scratch_shapes=[pltpu.VMEM((CAP, D), jnp.float32)],
        compiler_params=pltpu.CompilerParams(
            dimension_semantics=("parallel", "arbitrary")),
    )(si2, x, w1, w2, gs2)


def _combine_body(x_hbm, yg_hbm, d1_hbm, d2_hbm, out_hbm,
                  d1a, d2a, d1b, d2b, y1a, y2a, xa, y1b, y2b, xb,
                  gsem, xsem, wsem):
    wid = lax.axis_index("c") * NS + lax.axis_index("s")
    tbase = wid * TOK_PER_TILE
    nck = TOK_PER_TILE // TCHUNK
    d1s = (d1a, d1b)
    d2s = (d2a, d2b)
    y1s = (y1a, y1b)
    y2s = (y2a, y2b)
    xs = (xa, xb)

    def issue(c, s):
        sl = pl.ds(tbase + c * TCHUNK, TCHUNK)
        pltpu.sync_copy(d1_hbm.at[sl], d1s[s])
        pltpu.sync_copy(d2_hbm.at[sl], d2s[s])
        return (pltpu.async_copy(yg_hbm.at[d1s[s]], y1s[s], gsem),
                pltpu.async_copy(yg_hbm.at[d2s[s]], y2s[s], gsem),
                pltpu.async_copy(x_hbm.at[sl], xs[s], xsem))

    dma = [None] * nck
    w = [None] * nck
    dma[0] = issue(0, 0)
    for c in range(nck):
        for t in dma[c]:
            t.wait()
        if c + 1 < nck:
            if c - 1 >= 0:
                w[c - 1].wait()
            dma[c + 1] = issue(c + 1, (c + 1) % 2)
        s = c % 2
        xv, y1, y2 = xs[s], y1s[s], y2s[s]
        for t in range(TCHUNK):
            def add(j, carry, t=t, xv=xv, y1=y1, y2=y2):
                for u in range(4):
                    sl = pl.ds(j * 64 + u * 16, 16)
                    xv[t, sl] = xv[t, sl] + y1[t, sl] + y2[t, sl]
                return carry
            lax.fori_loop(0, D // 64, add, 0)
        w[c] = pltpu.async_copy(xv, out_hbm.at[pl.ds(tbase + c * TCHUNK,
                                                     TCHUNK)], wsem)
    w[nck - 2].wait()
    w[nck - 1].wait()


def _combine(x, yg, d1, d2):
    kern = pl.kernel(
        _combine_body,
        out_type=jax.ShapeDtypeStruct((T, D), jnp.float32),
        mesh=plsc.VectorSubcoreMesh(**_SC_MESH),
        compiler_params=_SC_PARAMS,
        scratch_types=[
            pltpu.VMEM((TCHUNK,), jnp.int32),
            pltpu.VMEM((TCHUNK,), jnp.int32),
            pltpu.VMEM((TCHUNK,), jnp.int32),
            pltpu.VMEM((TCHUNK,), jnp.int32),
            pltpu.VMEM((TCHUNK, D), jnp.float32),
            pltpu.VMEM((TCHUNK, D), jnp.float32),
            pltpu.VMEM((TCHUNK, D), jnp.float32),
            pltpu.VMEM((TCHUNK, D), jnp.float32),
            pltpu.VMEM((TCHUNK, D), jnp.float32),
            pltpu.VMEM((TCHUNK, D), jnp.float32),
            pltpu.SemaphoreType.DMA,
            pltpu.SemaphoreType.DMA,
            pltpu.SemaphoreType.DMA,
        ],
    )
    return kern(x, yg, d1, d2)


def _moe_layer(h, router_w, w1, w2, l):
    d1, d2, v1, v2, g1, g2 = _router(h, router_w, l)
    si, gs = _build_idx(d1, d2, v1, v2, g1, g2)
    yg = _gemm(si, h, w1, w2, gs, l)
    return _combine(h, yg, d1, d2)


def kernel(x, router_w, w1, w2):
    h = x
    for l in range(2):
        h = _moe_layer(h, router_w, w1, w2, l)
    return h


# speedup vs baseline: 2.1117x; 1.0229x over previous
import functools

import jax
import jax.numpy as jnp
from jax import lax
from jax.experimental import pallas as pl
from jax.experimental.pallas import tpu as pltpu
from jax.experimental.pallas import tpu_sc as plsc

E = 8
D = 1024
F = 4096
T = 2048
CAP = 640
NSLOT = E * CAP
FB = 2048
NC, NS = 2, 16
NW = NC * NS
ROWS_PER_TILE = NSLOT // NW
RCHUNK = 16
NBUF = 4
TOK_PER_TILE = T // NW
TCHUNK = 16

_SC_MESH = dict(core_axis_name="c", subcore_axis_name="s",
                num_cores=NC, num_subcores=NS)
_SC_PARAMS = pltpu.CompilerParams(needs_layout_passes=False)


def _router_body(h_ref, rw_ref, d1_ref, d2_ref, v1_ref, v2_ref,
                 g1_ref, g2_ref):
    x = h_ref[...]
    rw = rw_ref[0]
    logits = jnp.dot(x, rw, preferred_element_type=jnp.float32)
    m = jnp.max(logits, axis=1, keepdims=True)
    z = jnp.exp(logits - m)
    p = z / jnp.sum(z, axis=1, keepdims=True)
    lane = lax.broadcasted_iota(jnp.int32, (T, E), 1).astype(jnp.float32)
    BIG = jnp.float32(1e9)
    p1 = jnp.max(p, axis=1, keepdims=True)
    i1 = jnp.min(jnp.where(p == p1, lane, BIG), axis=1, keepdims=True)
    pm = jnp.where(lane == i1, -1.0, p)
    p2 = jnp.max(pm, axis=1, keepdims=True)
    i2 = jnp.min(jnp.where(pm == p2, lane, BIG), axis=1, keepdims=True)
    s = p1 + p2
    oh1 = lane == i1
    oh2 = lane == i2
    cnt = oh1.astype(jnp.float32) + oh2.astype(jnp.float32)
    r = lax.broadcasted_iota(jnp.int32, (128, 128), 0)
    c = lax.broadcasted_iota(jnp.int32, (128, 128), 1)
    ltri = (r > c).astype(jnp.float32)
    tot = jnp.zeros((1, E), jnp.float32)
    blocks = []
    for b in range(T // 128):
        cb = cnt[b * 128:(b + 1) * 128, :]
        blocks.append(jnp.dot(ltri, cb, preferred_element_type=jnp.float32)
                      + tot)
        tot = tot + jnp.sum(cb, axis=0, keepdims=True)
    excl = jnp.concatenate(blocks, axis=0)
    pos1 = jnp.sum(jnp.where(oh1, excl, 0.0), axis=1, keepdims=True)
    pos2 = jnp.sum(jnp.where(oh2, excl, 0.0), axis=1, keepdims=True)
    eidx = lax.broadcasted_iota(jnp.int32, (1, E), 1).astype(jnp.float32)
    free_e = jnp.min(jnp.where(tot < CAP, eidx, BIG), axis=1, keepdims=True)
    free_slot = free_e * CAP + (CAP - 1)
    val1 = pos1 < CAP
    val2 = pos2 < CAP
    d1_ref[...] = jnp.where(val1, i1 * CAP + pos1, free_slot).astype(jnp.int32)
    d2_ref[...] = jnp.where(val2, i2 * CAP + pos2, free_slot).astype(jnp.int32)
    v1_ref[...] = val1.astype(jnp.int32)
    v2_ref[...] = val2.astype(jnp.int32)
    g1_ref[...] = p1 / s
    g2_ref[...] = p2 / s


def _router(h, rw_full, l):
    col_i = jax.ShapeDtypeStruct((T, 1), jnp.int32)
    col_f = jax.ShapeDtypeStruct((T, 1), jnp.float32)
    outs = pl.pallas_call(
        _router_body,
        grid=(1,),
        in_specs=[
            pl.BlockSpec((T, D), lambda i: (0, 0)),
            pl.BlockSpec((1, D, E), lambda i: (l, 0, 0)),
        ],
        out_specs=tuple(pl.BlockSpec((T, 1), lambda i: (0, 0))
                        for _ in range(6)),
        out_shape=(col_i, col_i, col_i, col_i, col_f, col_f),
    )(h, rw_full)
    return tuple(o.reshape(T) for o in outs)


def _build_body(d1_hbm, d2_hbm, v1_hbm, v2_hbm, g1_hbm, g2_hbm,
                si_hbm, gs_hbm,
                d1v, d2v, v1v, v2v, g1v, g2v, siv, gsv):
    cid = lax.axis_index("c")
    sid = lax.axis_index("s")

    @pl.when((sid == 0) & (cid == 0))
    def _():
        pltpu.sync_copy(d1_hbm, d1v)
        pltpu.sync_copy(d2_hbm, d2v)
        pltpu.sync_copy(v1_hbm, v1v)
        pltpu.sync_copy(v2_hbm, v2v)
        pltpu.sync_copy(g1_hbm, g1v)
        pltpu.sync_copy(g2_hbm, g2v)

        def zero(i, carry):
            sl = pl.ds(i * 16, 16)
            siv[sl] = jnp.zeros((16,), jnp.int32)
            gsv[sl] = jnp.zeros((16,), jnp.float32)
            return carry
        lax.fori_loop(0, NSLOT // 16, zero, 0)

        def scat(i, carry):
            sl = pl.ds(i * 16, 16)
            tok = lax.iota(jnp.int32, 16) + i * 16
            for dv, vv, gv in ((d1v, v1v, g1v), (d2v, v2v, g2v)):
                d = dv[sl]
                msk = vv[sl] > 0
                plsc.store_scatter(siv, [d], tok, mask=msk)
                plsc.store_scatter(gsv, [d], gv[sl], mask=msk)
            return carry
        lax.fori_loop(0, T // 16, scat, 0)

        pltpu.sync_copy(siv, si_hbm)
        pltpu.sync_copy(gsv, gs_hbm)


def _build_idx(d1, d2, v1, v2, g1, g2):
    kern = pl.kernel(
        _build_body,
        out_type=(jax.ShapeDtypeStruct((NSLOT,), jnp.int32),
                  jax.ShapeDtypeStruct((NSLOT,), jnp.float32)),
        mesh=plsc.VectorSubcoreMesh(**_SC_MESH),
        compiler_params=_SC_PARAMS,
        scratch_types=[
            pltpu.VMEM((T,), jnp.int32), pltpu.VMEM((T,), jnp.int32),
            pltpu.VMEM((T,), jnp.int32), pltpu.VMEM((T,), jnp.int32),
            pltpu.VMEM((T,), jnp.float32), pltpu.VMEM((T,), jnp.float32),
            pltpu.VMEM((NSLOT,), jnp.int32), pltpu.VMEM((NSLOT,), jnp.float32),
        ],
    )
    return kern(d1, d2, v1, v2, g1, g2)


KB = 256


def _gemm_body(si_ref, x_ref, w1_ref, w2_ref, gs_ref, out_ref, xe_s):
    f = pl.program_id(1)

    @pl.when(f == 0)
    def _():
        si_col = si_ref[...]
        acc = jnp.zeros((CAP, D), jnp.float32)
        for kb in range(T // KB):
            tok = lax.broadcasted_iota(jnp.int32, (CAP, KB), 1) + kb * KB
            ce = (si_col == tok).astype(jnp.float32)
            acc = acc + jnp.dot(ce, x_ref[kb * KB:(kb + 1) * KB, :],
                                preferred_element_type=jnp.float32)
        xe_s[...] = acc

    h1 = jnp.dot(xe_s[...], w1_ref[0, 0], preferred_element_type=jnp.float32)
    h1 = jax.nn.gelu(h1)
    y = jnp.dot(h1, w2_ref[0, 0], preferred_element_type=jnp.float32)
    y = y * gs_ref[...]

    @pl.when(f == 0)
    def _():
        out_ref[...] = y

    @pl.when(f > 0)
    def _():
        out_ref[...] += y


def _gemm(si, x, w1, w2, gs, l):
    si2 = si.reshape(NSLOT, 1)
    gs2 = gs.reshape(NSLOT, 1)
    return pl.pallas_call(
        _gemm_body,
        grid=(E, F // FB),
        in_specs=[
            pl.BlockSpec((CAP, 1), lambda e, f: (e, 0)),
            pl.BlockSpec((T, D), lambda e, f: (0, 0)),
            pl.BlockSpec((1, 1, D, FB), lambda e, f: (l, e, 0, f)),
            pl.BlockSpec((1, 1, FB, D), lambda e, f: (l, e, f, 0)),
            pl.BlockSpec((CAP, 1), lambda e, f: (e, 0)),
        ],
        out_specs=pl.BlockSpec((CAP, D), lambda e, f: (e, 0)),
        out_shape=jax.ShapeDtypeStruct((NSLOT, D), jnp.float32),
        scratch_shapes=[pltpu.VMEM((CAP, D), jnp.float32)],
        compiler_params=pltpu.CompilerParams(
            dimension_semantics=("parallel", "arbitrary")),
    )(si2, x, w1, w2, gs2)


def _combine_body(x_hbm, yg_hbm, d1_hbm, d2_hbm, out_hbm,
                  d1a, d2a, d1b, d2b, y1a, y2a, xa, y1b, y2b, xb,
                  gsem, xsem, wsem):
    wid = lax.axis_index("c") * NS + lax.axis_index("s")
    tbase = wid * TOK_PER_TILE
    nck = TOK_PER_TILE // TCHUNK
    d1s = (d1a, d1b)
    d2s = (d2a, d2b)
    y1s = (y1a, y1b)
    y2s = (y2a, y2b)
    xs = (xa, xb)

    def issue(c, s):
        sl = pl.ds(tbase + c * TCHUNK, TCHUNK)
        pltpu.sync_copy(d1_hbm.at[sl], d1s[s])
        pltpu.sync_copy(d2_hbm.at[sl], d2s[s])
        return (pltpu.async_copy(yg_hbm.at[d1s[s]], y1s[s], gsem),
                pltpu.async_copy(yg_hbm.at[d2s[s]], y2s[s], gsem),
                pltpu.async_copy(x_hbm.at[sl], xs[s], xsem))

    dma = [None] * nck
    w = [None] * nck
    dma[0] = issue(0, 0)
    for c in range(nck):
        for t in dma[c]:
            t.wait()
        if c + 1 < nck:
            if c - 1 >= 0:
                w[c - 1].wait()
            dma[c + 1] = issue(c + 1, (c + 1) % 2)
        s = c % 2
        xv, y1, y2 = xs[s], y1s[s], y2s[s]
        for t in range(TCHUNK):
            def add(j, carry, t=t, xv=xv, y1=y1, y2=y2):
                for u in range(4):
                    sl = pl.ds(j * 64 + u * 16, 16)
                    xv[t, sl] = xv[t, sl] + y1[t, sl] + y2[t, sl]
                return carry
            lax.fori_loop(0, D // 64, add, 0)
        w[c] = pltpu.async_copy(xv, out_hbm.at[pl.ds(tbase + c * TCHUNK,
                                                     TCHUNK)], wsem)
    w[nck - 2].wait()
    w[nck - 1].wait()


def _combine(x, yg, d1, d2):
    kern = pl.kernel(
        _combine_body,
        out_type=jax.ShapeDtypeStruct((T, D), jnp.float32),
        mesh=plsc.VectorSubcoreMesh(**_SC_MESH),
        compiler_params=_SC_PARAMS,
        scratch_types=[
            pltpu.VMEM((TCHUNK,), jnp.int32),
            pltpu.VMEM((TCHUNK,), jnp.int32),
            pltpu.VMEM((TCHUNK,), jnp.int32),
            pltpu.VMEM((TCHUNK,), jnp.int32),
            pltpu.VMEM((TCHUNK, D), jnp.float32),
            pltpu.VMEM((TCHUNK, D), jnp.float32),
            pltpu.VMEM((TCHUNK, D), jnp.float32),
            pltpu.VMEM((TCHUNK, D), jnp.float32),
            pltpu.VMEM((TCHUNK, D), jnp.float32),
            pltpu.VMEM((TCHUNK, D), jnp.float32),
            pltpu.SemaphoreType.DMA,
            pltpu.SemaphoreType.DMA,
            pltpu.SemaphoreType.DMA,
        ],
    )
    return kern(x, yg, d1, d2)


def _moe_layer(h, router_w, w1, w2, l):
    d1, d2, v1, v2, g1, g2 = _router(h, router_w, l)
    si, gs = _build_idx(d1, d2, v1, v2, g1, g2)
    yg = _gemm(si, h, w1, w2, gs, l)
    return _combine(h, yg, d1, d2)


def kernel(x, router_w, w1, w2):
    h = x
    for l in range(2):
        h = _moe_layer(h, router_w, w1, w2, l)
    return h


# speedup vs baseline: 2.1125x; 1.0004x over previous
import jax
import jax.numpy as jnp
from jax import lax
from jax.experimental import pallas as pl
from jax.experimental.pallas import tpu as pltpu
from jax.experimental.pallas import tpu_sc as plsc

E = 8
D = 1024
F = 4096
T = 2048
CAP = 640
NSLOT = E * CAP
FB = 2048
NC, NS = 2, 16
NW = NC * NS
TOK_PER_TILE = T // NW
TCHUNK = 16

_SC_MESH = dict(core_axis_name="c", subcore_axis_name="s",
                num_cores=NC, num_subcores=NS)
_SC_PARAMS = pltpu.CompilerParams(needs_layout_passes=False)


def _router_body(h_ref, rw_ref, d1_ref, d2_ref, v1_ref, v2_ref,
                 g1_ref, g2_ref):
    x = h_ref[...]
    rw = rw_ref[0]
    logits = jnp.dot(x, rw, preferred_element_type=jnp.float32)
    m = jnp.max(logits, axis=1, keepdims=True)
    z = jnp.exp(logits - m)
    p = z / jnp.sum(z, axis=1, keepdims=True)
    lane = lax.broadcasted_iota(jnp.int32, (T, E), 1).astype(jnp.float32)
    BIG = jnp.float32(1e9)
    p1 = jnp.max(p, axis=1, keepdims=True)
    i1 = jnp.min(jnp.where(p == p1, lane, BIG), axis=1, keepdims=True)
    pm = jnp.where(lane == i1, -1.0, p)
    p2 = jnp.max(pm, axis=1, keepdims=True)
    i2 = jnp.min(jnp.where(pm == p2, lane, BIG), axis=1, keepdims=True)
    s = p1 + p2
    oh1 = lane == i1
    oh2 = lane == i2
    cnt = oh1.astype(jnp.float32) + oh2.astype(jnp.float32)
    r = lax.broadcasted_iota(jnp.int32, (128, 128), 0)
    c = lax.broadcasted_iota(jnp.int32, (128, 128), 1)
    ltri = (r > c).astype(jnp.float32)
    tot = jnp.zeros((1, E), jnp.float32)
    blocks = []
    for b in range(T // 128):
        cb = cnt[b * 128:(b + 1) * 128, :]
        blocks.append(jnp.dot(ltri, cb, preferred_element_type=jnp.float32)
                      + tot)
        tot = tot + jnp.sum(cb, axis=0, keepdims=True)
    excl = jnp.concatenate(blocks, axis=0)
    pos1 = jnp.sum(jnp.where(oh1, excl, 0.0), axis=1, keepdims=True)
    pos2 = jnp.sum(jnp.where(oh2, excl, 0.0), axis=1, keepdims=True)
    eidx = lax.broadcasted_iota(jnp.int32, (1, E), 1).astype(jnp.float32)
    free_e = jnp.min(jnp.where(tot < CAP, eidx, BIG), axis=1, keepdims=True)
    free_slot = free_e * CAP + (CAP - 1)
    val1 = pos1 < CAP
    val2 = pos2 < CAP
    d1_ref[...] = jnp.where(val1, i1 * CAP + pos1, free_slot).astype(jnp.int32)
    d2_ref[...] = jnp.where(val2, i2 * CAP + pos2, free_slot).astype(jnp.int32)
    v1_ref[...] = val1.astype(jnp.int32)
    v2_ref[...] = val2.astype(jnp.int32)
    g1_ref[...] = p1 / s
    g2_ref[...] = p2 / s


def _router(h, rw_full, l):
    col_i = jax.ShapeDtypeStruct((T, 1), jnp.int32)
    col_f = jax.ShapeDtypeStruct((T, 1), jnp.float32)
    outs = pl.pallas_call(
        _router_body,
        grid=(1,),
        in_specs=[
            pl.BlockSpec((T, D), lambda i: (0, 0)),
            pl.BlockSpec((1, D, E), lambda i: (l, 0, 0)),
        ],
        out_specs=tuple(pl.BlockSpec((T, 1), lambda i: (0, 0))
                        for _ in range(6)),
        out_shape=(col_i, col_i, col_i, col_i, col_f, col_f),
    )(h, rw_full)
    return tuple(o.reshape(T) for o in outs)


def _build_body(d1_hbm, d2_hbm, v1_hbm, v2_hbm, g1_hbm, g2_hbm,
                si_hbm, gs_hbm,
                d1v, d2v, v1v, v2v, g1v, g2v, siv, gsv):
    cid = lax.axis_index("c")
    sid = lax.axis_index("s")

    @pl.when((sid == 0) & (cid == 0))
    def _():
        pltpu.sync_copy(d1_hbm, d1v)
        pltpu.sync_copy(d2_hbm, d2v)
        pltpu.sync_copy(v1_hbm, v1v)
        pltpu.sync_copy(v2_hbm, v2v)
        pltpu.sync_copy(g1_hbm, g1v)
        pltpu.sync_copy(g2_hbm, g2v)

        def zero(i, carry):
            sl = pl.ds(i * 16, 16)
            siv[sl] = jnp.zeros((16,), jnp.int32)
            gsv[sl] = jnp.zeros((16,), jnp.float32)
            return carry
        lax.fori_loop(0, NSLOT // 16, zero, 0)

        def scat(i, carry):
            sl = pl.ds(i * 16, 16)
            tok = lax.iota(jnp.int32, 16) + i * 16
            for dv, vv, gv in ((d1v, v1v, g1v), (d2v, v2v, g2v)):
                d = dv[sl]
                msk = vv[sl] > 0
                plsc.store_scatter(siv, [d], tok, mask=msk)
                plsc.store_scatter(gsv, [d], gv[sl], mask=msk)
            return carry
        lax.fori_loop(0, T // 16, scat, 0)

        pltpu.sync_copy(siv, si_hbm)
        pltpu.sync_copy(gsv, gs_hbm)


def _build_idx(d1, d2, v1, v2, g1, g2):
    kern = pl.kernel(
        _build_body,
        out_type=(jax.ShapeDtypeStruct((NSLOT,), jnp.int32),
                  jax.ShapeDtypeStruct((NSLOT,), jnp.float32)),
        mesh=plsc.VectorSubcoreMesh(**_SC_MESH),
        compiler_params=_SC_PARAMS,
        scratch_types=[
            pltpu.VMEM((T,), jnp.int32), pltpu.VMEM((T,), jnp.int32),
            pltpu.VMEM((T,), jnp.int32), pltpu.VMEM((T,), jnp.int32),
            pltpu.VMEM((T,), jnp.float32), pltpu.VMEM((T,), jnp.float32),
            pltpu.VMEM((NSLOT,), jnp.int32), pltpu.VMEM((NSLOT,), jnp.float32),
        ],
    )
    return kern(d1, d2, v1, v2, g1, g2)


KB = 256


def _gemm_body(si_ref, x_ref, w1_ref, w2_ref, gs_ref, out_ref, xe_s):
    f = pl.program_id(1)

    @pl.when(f == 0)
    def _():
        si_col = si_ref[...]
        acc = jnp.zeros((CAP, D), jnp.float32)
        for kb in range(T // KB):
            tok = lax.broadcasted_iota(jnp.int32, (CAP, KB), 1) + kb * KB
            ce = (si_col == tok).astype(jnp.float32)
            acc = acc + jnp.dot(ce, x_ref[kb * KB:(kb + 1) * KB, :],
                                preferred_element_type=jnp.float32)
        xe_s[...] = acc

    h1 = jnp.dot(xe_s[...], w1_ref[0, 0], preferred_element_type=jnp.float32)
    h1 = jax.nn.gelu(h1)
    y = jnp.dot(h1, w2_ref[0, 0], preferred_element_type=jnp.float32)
    y = y * gs_ref[...]

    @pl.when(f == 0)
    def _():
        out_ref[...] = y

    @pl.when(f > 0)
    def _():
        out_ref[...] += y


def _gemm(si, x, w1, w2, gs, l):
    si2 = si.reshape(NSLOT, 1)
    gs2 = gs.reshape(NSLOT, 1)
    return pl.pallas_call(
        _gemm_body,
        grid=(E, F // FB),
        in_specs=[
            pl.BlockSpec((CAP, 1), lambda e, f: (e, 0)),
            pl.BlockSpec((T, D), lambda e, f: (0, 0)),
            pl.BlockSpec((1, 1, D, FB), lambda e, f: (l, e, 0, f)),
            pl.BlockSpec((1, 1, FB, D), lambda e, f: (l, e, f, 0)),
            pl.BlockSpec((CAP, 1), lambda e, f: (e, 0)),
        ],
        out_specs=pl.BlockSpec((CAP, D), lambda e, f: (e, 0)),
        out_shape=jax.ShapeDtypeStruct((NSLOT, D), jnp.float32),
        scratch_shapes=[pltpu.VMEM((CAP, D), jnp.float32)],
        compiler_params=pltpu.CompilerParams(
            dimension_semantics=("parallel", "arbitrary")),
    )(si2, x, w1, w2, gs2)


def _combine_body(x_hbm, yg_hbm, d1_hbm, d2_hbm, out_hbm,
                  d1a, d2a, d1b, d2b, y1a, y2a, xa, y1b, y2b, xb,
                  gsem, xsem, wsem):
    wid = lax.axis_index("c") * NS + lax.axis_index("s")
    tbase = wid * TOK_PER_TILE
    nck = TOK_PER_TILE // TCHUNK
    d1s = (d1a, d1b)
    d2s = (d2a, d2b)
    y1s = (y1a, y1b)
    y2s = (y2a, y2b)
    xs = (xa, xb)

    def issue(c, s):
        sl = pl.ds(tbase + c * TCHUNK, TCHUNK)
        pltpu.sync_copy(d1_hbm.at[sl], d1s[s])
        pltpu.sync_copy(d2_hbm.at[sl], d2s[s])
        return (pltpu.async_copy(yg_hbm.at[d1s[s]], y1s[s], gsem),
                pltpu.async_copy(yg_hbm.at[d2s[s]], y2s[s], gsem),
                pltpu.async_copy(x_hbm.at[sl], xs[s], xsem))

    dma = [None] * nck
    w = [None] * nck
    dma[0] = issue(0, 0)
    for c in range(nck):
        for t in dma[c]:
            t.wait()
        if c + 1 < nck:
            if c - 1 >= 0:
                w[c - 1].wait()
            dma[c + 1] = issue(c + 1, (c + 1) % 2)
        s = c % 2
        xv, y1, y2 = xs[s], y1s[s], y2s[s]
        for t in range(TCHUNK):
            def add(j, carry, t=t, xv=xv, y1=y1, y2=y2):
                for u in range(4):
                    sl = pl.ds(j * 64 + u * 16, 16)
                    xv[t, sl] = xv[t, sl] + y1[t, sl] + y2[t, sl]
                return carry
            lax.fori_loop(0, D // 64, add, 0)
        w[c] = pltpu.async_copy(xv, out_hbm.at[pl.ds(tbase + c * TCHUNK,
                                                     TCHUNK)], wsem)
    w[nck - 2].wait()
    w[nck - 1].wait()


def _combine(x, yg, d1, d2):
    kern = pl.kernel(
        _combine_body,
        out_type=jax.ShapeDtypeStruct((T, D), jnp.float32),
        mesh=plsc.VectorSubcoreMesh(**_SC_MESH),
        compiler_params=_SC_PARAMS,
        scratch_types=[
            pltpu.VMEM((TCHUNK,), jnp.int32),
            pltpu.VMEM((TCHUNK,), jnp.int32),
            pltpu.VMEM((TCHUNK,), jnp.int32),
            pltpu.VMEM((TCHUNK,), jnp.int32),
            pltpu.VMEM((TCHUNK, D), jnp.float32),
            pltpu.VMEM((TCHUNK, D), jnp.float32),
            pltpu.VMEM((TCHUNK, D), jnp.float32),
            pltpu.VMEM((TCHUNK, D), jnp.float32),
            pltpu.VMEM((TCHUNK, D), jnp.float32),
            pltpu.VMEM((TCHUNK, D), jnp.float32),
            pltpu.SemaphoreType.DMA,
            pltpu.SemaphoreType.DMA,
            pltpu.SemaphoreType.DMA,
        ],
    )
    return kern(x, yg, d1, d2)


def _moe_layer(h, router_w, w1, w2, l):
    d1, d2, v1, v2, g1, g2 = _router(h, router_w, l)
    si, gs = _build_idx(d1, d2, v1, v2, g1, g2)
    yg = _gemm(si, h, w1, w2, gs, l)
    return _combine(h, yg, d1, d2)


def kernel(x, router_w, w1, w2):
    h = x
    for l in range(2):
        h = _moe_layer(h, router_w, w1, w2, l)
    return h
